# baseline (speedup 1.0000x reference)
import functools

import jax
import jax.numpy as jnp
from jax import lax
from jax.experimental import pallas as pl
from jax.experimental.pallas import tpu as pltpu
from jax.experimental.pallas import tpu_sc as plsc

F32 = jnp.float32
N_NODES = 10000
N_EDGES = 320000
N_HEADS = 8
D_HEAD = 16
H = 128

EBLK = 2000
NBLK = 2000
NW = 32
EPW = N_EDGES // NW
C = 80
NCH = EPW // C
EPW2 = N_EDGES // 16
C2 = 40
NCH2 = EPW2 // C2
STR = 624


def _ln(x, g, b):
    mu = jnp.mean(x, axis=-1, keepdims=True)
    var = jnp.mean((x - mu) ** 2, axis=-1, keepdims=True)
    return (x - mu) / jnp.sqrt(var + 1e-5) * g + b


def _head_masks():
    i = lax.broadcasted_iota(jnp.int32, (H, N_HEADS), 0)
    j = lax.broadcasted_iota(jnp.int32, (H, N_HEADS), 1)
    hm8 = (i // D_HEAD == j).astype(F32)
    ex8 = hm8.T
    return hm8, ex8


def _node_init_body(x_ref, lap_ref, wh, bh, wl, bl, wq, bq, wk, bk, wv, bv,
                    h_ref, q_ref, k_ref, v_ref):
    h = (jnp.dot(x_ref[...], wh[...], preferred_element_type=F32) + bh[...]
         + jnp.dot(lap_ref[...], wl[...], preferred_element_type=F32) + bl[...])
    h_ref[...] = h
    q_ref[...] = jnp.dot(h, wq[...], preferred_element_type=F32) + bq[...]
    k_ref[...] = jnp.dot(h, wk[...], preferred_element_type=F32) + bk[...]
    v_ref[...] = jnp.dot(h, wv[...], preferred_element_type=F32) + bv[...]


def _full(w):
    return pl.BlockSpec(w.shape, lambda i: (0,) * w.ndim)


def _rows(d, blk):
    return pl.BlockSpec((blk, d), lambda i: (i, 0))


def _node_init(x, lap, p):
    wh, bh = p["linear_h"]["W"], p["linear_h"]["b"].reshape(1, -1)
    wl, bl = p["lap"]["W"], p["lap"]["b"].reshape(1, -1)
    l0 = p["layers"][0]
    args = (x, lap, wh, bh, wl, bl,
            l0["Q"]["W"], l0["Q"]["b"].reshape(1, -1),
            l0["K"]["W"], l0["K"]["b"].reshape(1, -1),
            l0["V"]["W"], l0["V"]["b"].reshape(1, -1))
    out = [jax.ShapeDtypeStruct((N_NODES, H), F32)] * 4
    return pl.pallas_call(
        _node_init_body,
        grid=(N_NODES // NBLK,),
        in_specs=[_rows(x.shape[1], NBLK), _rows(lap.shape[1], NBLK)]
                 + [_full(a) for a in args[2:]],
        out_specs=[_rows(H, NBLK)] * 4,
        out_shape=out,
    )(*args)


def _edge_body(first, last, *refs):
    it = iter(refs)
    e_ref = next(it)
    kg, qg, vg = next(it), next(it), next(it)
    wpe, bpe = next(it), next(it)
    if first:
        we, be = next(it), next(it)
    if not last:
        woe, boe = next(it), next(it)
        g1, b1 = next(it), next(it)
        wf1, bf1 = next(it), next(it)
        wf2, bf2 = next(it), next(it)
        g2, b2 = next(it), next(it)
    sv_ref, s16_ref = next(it), next(it)
    if not last:
        en_ref = next(it)

    if first:
        e = jnp.dot(e_ref[...], we[...], preferred_element_type=F32) + be[...]
    else:
        e = e_ref[...]
    E = jnp.dot(e, wpe[...], preferred_element_type=F32) + bpe[...]
    score = kg[...] * qg[...] * E * 0.25
    hm8, ex8 = _head_masks()
    hsum = jnp.dot(score, hm8, preferred_element_type=F32)
    s8 = jnp.exp(jnp.clip(hsum, -5.0, 5.0))
    s_b = jnp.dot(s8, ex8, preferred_element_type=F32)
    sv_ref[...] = s_b * vg[...]
    s16_ref[...] = jnp.concatenate([s8, jnp.zeros_like(s8)], axis=-1)
    if not last:
        e1 = _ln(e + jnp.dot(score, woe[...], preferred_element_type=F32)
                 + boe[...], g1[...], b1[...])
        f = jnp.maximum(jnp.dot(e1, wf1[...], preferred_element_type=F32)
                        + bf1[...], 0.0)
        e2 = e1 + jnp.dot(f, wf2[...], preferred_element_type=F32) + bf2[...]
        en_ref[...] = _ln(e2, g2[...], b2[...])


def _edge_pass(layer_idx, lp, e_in, kg, qg, vg, p):
    first = layer_idx == 0
    last = layer_idx == 3
    args = [e_in, kg, qg, vg,
            lp["proj_e"]["W"], lp["proj_e"]["b"].reshape(1, -1)]
    if first:
        args += [p["linear_e"]["W"], p["linear_e"]["b"].reshape(1, -1)]
    if not last:
        args += [lp["O_e"]["W"], lp["O_e"]["b"].reshape(1, -1),
                 lp["ln1_e_g"].reshape(1, -1), lp["ln1_e_b"].reshape(1, -1),
                 lp["ffn_e1"]["W"], lp["ffn_e1"]["b"].reshape(1, -1),
                 lp["ffn_e2"]["W"], lp["ffn_e2"]["b"].reshape(1, -1),
                 lp["ln2_e_g"].reshape(1, -1), lp["ln2_e_b"].reshape(1, -1)]
    out_shape = [jax.ShapeDtypeStruct((N_EDGES, H), F32),
                 jax.ShapeDtypeStruct((N_EDGES, 16), F32)]
    out_specs = [_rows(H, EBLK), _rows(16, EBLK)]
    if not last:
        out_shape.append(jax.ShapeDtypeStruct((N_EDGES, H), F32))
        out_specs.append(_rows(H, EBLK))
    return pl.pallas_call(
        functools.partial(_edge_body, first, last),
        grid=(N_EDGES // EBLK,),
        in_specs=[_rows(e_in.shape[1], EBLK)] + [_rows(H, EBLK)] * 3
                 + [_full(a) for a in args[4:]],
        out_specs=out_specs,
        out_shape=out_shape,
    )(*args)


def _node_body(last, *refs):
    it = iter(refs)
    h_ref = next(it)
    wv0, z0 = next(it), next(it)
    woh, boh = next(it), next(it)
    g1, b1 = next(it), next(it)
    wf1, bf1 = next(it), next(it)
    wf2, bf2 = next(it), next(it)
    g2, b2 = next(it), next(it)
    if not last:
        wq, bq, wk, bk, wv_, bv = (next(it), next(it), next(it), next(it),
                                   next(it), next(it))
    h_out = next(it)
    if not last:
        q_ref, k_ref, v_ref = next(it), next(it), next(it)

    h_att = wv0[...] / (z0[...] + 1e-6)
    h = h_ref[...]
    h1 = _ln(h + jnp.dot(h_att, woh[...], preferred_element_type=F32)
             + boh[...], g1[...], b1[...])
    f = jnp.maximum(jnp.dot(h1, wf1[...], preferred_element_type=F32)
                    + bf1[...], 0.0)
    hn = _ln(h1 + jnp.dot(f, wf2[...], preferred_element_type=F32) + bf2[...],
             g2[...], b2[...])
    h_out[...] = hn
    if not last:
        q_ref[...] = jnp.dot(hn, wq[...], preferred_element_type=F32) + bq[...]
        k_ref[...] = jnp.dot(hn, wk[...], preferred_element_type=F32) + bk[...]
        v_ref[...] = jnp.dot(hn, wv_[...], preferred_element_type=F32) + bv[...]


def _node_pass(layer_idx, lp, lp_next, h, wvz):
    last = layer_idx == 3
    args = [h, wvz[0], wvz[1],
            lp["O_h"]["W"], lp["O_h"]["b"].reshape(1, -1),
            lp["ln1_h_g"].reshape(1, -1), lp["ln1_h_b"].reshape(1, -1),
            lp["ffn_h1"]["W"], lp["ffn_h1"]["b"].reshape(1, -1),
            lp["ffn_h2"]["W"], lp["ffn_h2"]["b"].reshape(1, -1),
            lp["ln2_h_g"].reshape(1, -1), lp["ln2_h_b"].reshape(1, -1)]
    if not last:
        args += [lp_next["Q"]["W"], lp_next["Q"]["b"].reshape(1, -1),
                 lp_next["K"]["W"], lp_next["K"]["b"].reshape(1, -1),
                 lp_next["V"]["W"], lp_next["V"]["b"].reshape(1, -1)]
    n_out = 1 if last else 4
    return pl.pallas_call(
        functools.partial(_node_body, last),
        grid=(N_NODES // NBLK,),
        in_specs=[_rows(H, NBLK)] * 3 + [_full(a) for a in args[3:]],
        out_specs=[_rows(H, NBLK)] * n_out,
        out_shape=[jax.ShapeDtypeStruct((N_NODES, H), F32)] * n_out,
    )(*args)


def _sc_gather_body(k_hbm, q_hbm, v_hbm, src2, dst2, kg_hbm, qg_hbm, vg_hbm,
                    idx_s, idx_d,
                    bufk0, bufk1, bufq0, bufq1, bufv0, bufv1,
                    gsk0, gsk1, gsq0, gsq1, gsv0, gsv1,
                    wsk0, wsk1, wsq0, wsq1, wsv0, wsv1):
    cid = lax.axis_index("c")
    sid = lax.axis_index("s")
    wid = cid * 16 + sid
    pltpu.sync_copy(src2.at[wid], idx_s)
    pltpu.sync_copy(dst2.at[wid], idx_d)
    ebase = wid * EPW
    bufs = ((bufk0, bufq0, bufv0), (bufk1, bufq1, bufv1))
    gs = ((gsk0, gsq0, gsv0), (gsk1, gsq1, gsv1))
    ws = ((wsk0, wsq0, wsv0), (wsk1, wsq1, wsv1))

    def g_start(cc, b):
        pltpu.async_copy(k_hbm.at[idx_s.at[cc]], bufs[b][0], gs[b][0])
        pltpu.async_copy(q_hbm.at[idx_d.at[cc]], bufs[b][1], gs[b][1])
        pltpu.async_copy(v_hbm.at[idx_s.at[cc]], bufs[b][2], gs[b][2])

    def g_wait(cc, b):
        pltpu.make_async_copy(k_hbm.at[idx_s.at[cc]], bufs[b][0], gs[b][0]).wait()
        pltpu.make_async_copy(q_hbm.at[idx_d.at[cc]], bufs[b][1], gs[b][1]).wait()
        pltpu.make_async_copy(v_hbm.at[idx_s.at[cc]], bufs[b][2], gs[b][2]).wait()

    def w_descr(cc, b):
        row0 = pl.multiple_of(ebase + cc * C, 8)
        return ((bufs[b][0], kg_hbm.at[pl.ds(row0, C), :], ws[b][0]),
                (bufs[b][1], qg_hbm.at[pl.ds(row0, C), :], ws[b][1]),
                (bufs[b][2], vg_hbm.at[pl.ds(row0, C), :], ws[b][2]))

    def w_start(cc, b):
        for s, d, sem in w_descr(cc, b):
            pltpu.async_copy(s, d, sem)

    def w_wait(cc, b):
        for s, d, sem in w_descr(cc, b):
            pltpu.make_async_copy(s, d, sem).wait()

    g_start(0, 0)

    def body(g, carry):
        for b in (0, 1):
            cc = 2 * g + b
            nb = (b + 1) % 2

            @pl.when(cc >= 1)
            def _():
                w_wait(cc - 1, nb)

            g_start(cc + 1, nb)
            g_wait(cc, b)
            w_start(cc, b)
        return carry

    lax.fori_loop(0, (NCH - 1) // 2, body, 0)
    w_wait(NCH - 2, 1)
    g_wait(NCH - 1, 0)
    w_start(NCH - 1, 0)
    w_wait(NCH - 1, 0)


IW = 50


def _lane_bcast(vec, lane):
    idx = jnp.full((16, 1), lane, jnp.int32)
    return lax.gather(
        vec, idx,
        lax.GatherDimensionNumbers(offset_dims=(), collapsed_slice_dims=(0,),
                                   start_index_map=(0,)),
        slice_sizes=(1,), mode=lax.GatherScatterMode.PROMISE_IN_BOUNDS)


def _sc_scatter_body(sv_hbm, s16_hbm, dst4, zin_hbm, out_hbm,
                     idx_d, bufv0, bufv1, bufs0, bufs1, bufexp,
                     acc_sh, sem0, sem1):
    cid = lax.axis_index("c")
    sid = lax.axis_index("s")
    r0 = pl.multiple_of(sid * STR, 8)
    pltpu.sync_copy(zin_hbm.at[pl.ds(r0, STR), :], acc_sh.at[pl.ds(r0, STR), :])
    rem = N_NODES - 16 * STR

    @pl.when(sid == 0)
    def _():
        pltpu.sync_copy(zin_hbm.at[pl.ds(16 * STR, rem), :],
                        acc_sh.at[pl.ds(16 * STR, rem), :])

    plsc.subcore_barrier()
    ebase = sid * EPW2
    sems = (sem0, sem1)

    def mkpipe(src_hbm, bufs, expand):
        def r_start(cc, b):
            row0 = pl.multiple_of(ebase + cc * C2, 8)
            pltpu.async_copy(src_hbm.at[pl.ds(row0, C2), :], bufs[b], sems[b])

        def r_wait(cc, b):
            row0 = pl.multiple_of(ebase + cc * C2, 8)
            pltpu.make_async_copy(src_hbm.at[pl.ds(row0, C2), :], bufs[b],
                                  sems[b]).wait()

        def run():
            r_start(0, 0)

            def body(g, carry):
                for b in (0, 1):
                    cc = 2 * g + b
                    nb = (b + 1) % 2

                    @pl.when(lax.rem(cc, IW) == 0)
                    def _():
                        pltpu.sync_copy(dst4.at[sid, lax.div(cc, IW)], idx_d)

                    @pl.when(cc + 1 < NCH2)
                    def _():
                        r_start(cc + 1, nb)

                    r_wait(cc, b)
                    if expand:
                        lanes = lax.iota(jnp.int32, 16)

                        def erow(e, carry2):
                            srow = bufs[b][e, :]
                            for hh in range(N_HEADS):
                                bufexp[e, pl.ds(hh * D_HEAD, D_HEAD)] = (
                                    _lane_bcast(srow, hh))
                            return carry2

                        lax.fori_loop(0, C2, erow, 0)
                        src_buf = bufexp
                    else:
                        src_buf = bufs[b]
                    pltpu.sync_copy(src_buf,
                                    acc_sh.at[idx_d.at[lax.rem(cc, IW)]],
                                    add=True)
                return carry

            lax.fori_loop(0, NCH2 // 2, body, 0)
        return run

    @pl.when(cid == 0)
    def _():
        mkpipe(sv_hbm, (bufv0, bufv1), False)()

    @pl.when(cid == 1)
    def _():
        mkpipe(s16_hbm, (bufs0, bufs1), True)()

    plsc.subcore_barrier()
    pltpu.sync_copy(acc_sh.at[pl.ds(r0, STR), :],
                    out_hbm.at[cid, pl.ds(r0, STR), :])

    @pl.when(sid == 0)
    def _():
        pltpu.sync_copy(acc_sh.at[pl.ds(16 * STR, rem), :],
                        out_hbm.at[cid, pl.ds(16 * STR, rem), :])


@functools.lru_cache(maxsize=None)
def _sc_kernels():
    mesh = plsc.VectorSubcoreMesh(core_axis_name="c", subcore_axis_name="s")
    gather = pl.kernel(
        _sc_gather_body, mesh=mesh,
        out_type=[jax.ShapeDtypeStruct((N_EDGES, H), F32)] * 3,
        scratch_types=[pltpu.VMEM((NCH, C), jnp.int32),
                       pltpu.VMEM((NCH, C), jnp.int32)]
                      + [pltpu.VMEM((C, H), F32) for _ in range(6)]
                      + [pltpu.SemaphoreType.DMA for _ in range(12)],
    )
    scatter = pl.kernel(
        _sc_scatter_body, mesh=mesh,
        out_type=[jax.ShapeDtypeStruct((2, N_NODES, H), F32)],
        scratch_types=[pltpu.VMEM((IW, C2), jnp.int32),
                       pltpu.VMEM((C2, H), F32),
                       pltpu.VMEM((C2, H), F32),
                       pltpu.VMEM((C2, 16), F32),
                       pltpu.VMEM((C2, 16), F32),
                       pltpu.VMEM((C2, H), F32),
                       pltpu.VMEM_SHARED((N_NODES, H), F32),
                       pltpu.SemaphoreType.DMA,
                       pltpu.SemaphoreType.DMA],
        compiler_params=pltpu.CompilerParams(needs_layout_passes=False),
    )
    return gather, scatter


def _gather_kqv(k, q, v, src2, dst2):
    return _sc_kernels()[0](k, q, v, src2, dst2)


def _scatter_segments(sv, sb, dst4):
    zin = jnp.zeros((N_NODES, H), F32)
    return _sc_kernels()[1](sv, sb, dst4, zin)[0]


def kernel(x, lap_pos_enc, edge_attr, params, edge_index):
    src2 = edge_index[0].reshape(NW, NCH, C)
    dst2 = edge_index[1].reshape(NW, NCH, C)
    dst4 = edge_index[1].reshape(16, NCH2 // IW, IW, C2)
    h, q, k, v = _node_init(x, lap_pos_enc, params)
    e = edge_attr
    for li in range(4):
        lp = params["layers"][li]
        lp_next = params["layers"][li + 1] if li < 3 else None
        kg, qg, vg = _gather_kqv(k, q, v, src2, dst2)
        outs = _edge_pass(li, lp, e, kg, qg, vg, params)
        if li < 3:
            sv, s16, e = outs
        else:
            sv, s16 = outs
        wvz = _scatter_segments(sv, s16, dst4)
        nouts = _node_pass(li, lp, lp_next, h, wvz)
        if li < 3:
            h, q, k, v = nouts
        else:
            h = nouts[0]
    return h


# speedup vs baseline: 1.2246x; 1.2246x over previous
import functools

import jax
import jax.numpy as jnp
from jax import lax
from jax.experimental import pallas as pl
from jax.experimental.pallas import tpu as pltpu
from jax.experimental.pallas import tpu_sc as plsc

F32 = jnp.float32
N_NODES = 10000
N_EDGES = 320000
N_HEADS = 8
D_HEAD = 16
H = 128

HALVES = 2
HE = N_EDGES // HALVES

EBLK = 2000
NBLK = 2000
NW = 32
EPW_G = HE // NW
C_G = 40
NCH_G = EPW_G // C_G
EPW_S = HE // 16
C_S = 80
NCH_S = EPW_S // C_S
IW = 25
STR = 624


def _ln(x, g, b):
    mu = jnp.mean(x, axis=-1, keepdims=True)
    var = jnp.mean((x - mu) ** 2, axis=-1, keepdims=True)
    return (x - mu) / jnp.sqrt(var + 1e-5) * g + b


def _head_masks():
    i = lax.broadcasted_iota(jnp.int32, (H, N_HEADS), 0)
    j = lax.broadcasted_iota(jnp.int32, (H, N_HEADS), 1)
    hm8 = (i // D_HEAD == j).astype(F32)
    return hm8, hm8.T


def _full(w):
    return pl.BlockSpec(w.shape, lambda i: (0,) * w.ndim)


def _rows(d, blk, off=0):
    if off:
        return pl.BlockSpec((blk, d), lambda i: (i + off, 0))
    return pl.BlockSpec((blk, d), lambda i: (i, 0))


def _node_init_body(x_ref, lap_ref, wh, bh, wl, bl, wq, bq, wk, bk, wv, bv,
                    h_ref, q_ref, k_ref, v_ref):
    h = (jnp.dot(x_ref[...], wh[...], preferred_element_type=F32) + bh[...]
         + jnp.dot(lap_ref[...], wl[...], preferred_element_type=F32) + bl[...])
    h_ref[...] = h
    q_ref[...] = jnp.dot(h, wq[...], preferred_element_type=F32) + bq[...]
    k_ref[...] = jnp.dot(h, wk[...], preferred_element_type=F32) + bk[...]
    v_ref[...] = jnp.dot(h, wv[...], preferred_element_type=F32) + bv[...]


def _node_init(x, lap, p):
    l0 = p["layers"][0]
    args = (x, lap,
            p["linear_h"]["W"], p["linear_h"]["b"].reshape(1, -1),
            p["lap"]["W"], p["lap"]["b"].reshape(1, -1),
            l0["Q"]["W"], l0["Q"]["b"].reshape(1, -1),
            l0["K"]["W"], l0["K"]["b"].reshape(1, -1),
            l0["V"]["W"], l0["V"]["b"].reshape(1, -1))
    return pl.pallas_call(
        _node_init_body,
        grid=(N_NODES // NBLK,),
        in_specs=[_rows(x.shape[1], NBLK), _rows(lap.shape[1], NBLK)]
                 + [_full(a) for a in args[2:]],
        out_specs=[_rows(H, NBLK)] * 4,
        out_shape=[jax.ShapeDtypeStruct((N_NODES, H), F32)] * 4,
    )(*args)


def _edge_body(first, last, *refs):
    it = iter(refs)
    e_ref = next(it)
    kg, qg, vg = next(it), next(it), next(it)
    wpe, bpe = next(it), next(it)
    if first:
        we, be = next(it), next(it)
    if not last:
        woe, boe = next(it), next(it)
        g1, b1 = next(it), next(it)
        wf1, bf1 = next(it), next(it)
        wf2, bf2 = next(it), next(it)
        g2, b2 = next(it), next(it)
    sv_ref, sb_ref = next(it), next(it)
    if not last:
        en_ref = next(it)

    if first:
        e = jnp.dot(e_ref[...], we[...], preferred_element_type=F32) + be[...]
    else:
        e = e_ref[...]
    E = jnp.dot(e, wpe[...], preferred_element_type=F32) + bpe[...]
    score = kg[...] * qg[...] * E * 0.25
    hm8, ex8 = _head_masks()
    hsum = jnp.dot(score, hm8, preferred_element_type=F32)
    s8 = jnp.exp(jnp.clip(hsum, -5.0, 5.0))
    s_b = jnp.dot(s8, ex8, preferred_element_type=F32)
    sv_ref[...] = s_b * vg[...]
    sb_ref[...] = s_b
    if not last:
        e1 = _ln(e + jnp.dot(score, woe[...], preferred_element_type=F32)
                 + boe[...], g1[...], b1[...])
        f = jnp.maximum(jnp.dot(e1, wf1[...], preferred_element_type=F32)
                        + bf1[...], 0.0)
        e2 = e1 + jnp.dot(f, wf2[...], preferred_element_type=F32) + bf2[...]
        en_ref[...] = _ln(e2, g2[...], b2[...])


def _edge_pass(layer_idx, half, lp, e_in, kg, qg, vg, p):
    first = layer_idx == 0
    last = layer_idx == 3
    off = half * (HE // EBLK) if first else 0
    args = [e_in, kg, qg, vg,
            lp["proj_e"]["W"], lp["proj_e"]["b"].reshape(1, -1)]
    if first:
        args += [p["linear_e"]["W"], p["linear_e"]["b"].reshape(1, -1)]
    if not last:
        args += [lp["O_e"]["W"], lp["O_e"]["b"].reshape(1, -1),
                 lp["ln1_e_g"].reshape(1, -1), lp["ln1_e_b"].reshape(1, -1),
                 lp["ffn_e1"]["W"], lp["ffn_e1"]["b"].reshape(1, -1),
                 lp["ffn_e2"]["W"], lp["ffn_e2"]["b"].reshape(1, -1),
                 lp["ln2_e_g"].reshape(1, -1), lp["ln2_e_b"].reshape(1, -1)]
    n_out = 2 if last else 3
    return pl.pallas_call(
        functools.partial(_edge_body, first, last),
        grid=(HE // EBLK,),
        in_specs=[_rows(e_in.shape[1], EBLK, off)] + [_rows(H, EBLK)] * 3
                 + [_full(a) for a in args[4:]],
        out_specs=[_rows(H, EBLK)] * n_out,
        out_shape=[jax.ShapeDtypeStruct((HE, H), F32)] * n_out,
    )(*args)


def _node_body(last, *refs):
    it = iter(refs)
    h_ref = next(it)
    wva, za, wvb, zb = next(it), next(it), next(it), next(it)
    woh, boh = next(it), next(it)
    g1, b1 = next(it), next(it)
    wf1, bf1 = next(it), next(it)
    wf2, bf2 = next(it), next(it)
    g2, b2 = next(it), next(it)
    if not last:
        wq, bq, wk, bk, wv_, bv = (next(it), next(it), next(it), next(it),
                                   next(it), next(it))
    h_out = next(it)
    if not last:
        q_ref, k_ref, v_ref = next(it), next(it), next(it)

    h_att = (wva[...] + wvb[...]) / (za[...] + zb[...] + 1e-6)
    h = h_ref[...]
    h1 = _ln(h + jnp.dot(h_att, woh[...], preferred_element_type=F32)
             + boh[...], g1[...], b1[...])
    f = jnp.maximum(jnp.dot(h1, wf1[...], preferred_element_type=F32)
                    + bf1[...], 0.0)
    hn = _ln(h1 + jnp.dot(f, wf2[...], preferred_element_type=F32) + bf2[...],
             g2[...], b2[...])
    h_out[...] = hn
    if not last:
        q_ref[...] = jnp.dot(hn, wq[...], preferred_element_type=F32) + bq[...]
        k_ref[...] = jnp.dot(hn, wk[...], preferred_element_type=F32) + bk[...]
        v_ref[...] = jnp.dot(hn, wv_[...], preferred_element_type=F32) + bv[...]


def _node_pass(layer_idx, lp, lp_next, h, wvza, wvzb):
    last = layer_idx == 3
    args = [h, wvza[0], wvza[1], wvzb[0], wvzb[1],
            lp["O_h"]["W"], lp["O_h"]["b"].reshape(1, -1),
            lp["ln1_h_g"].reshape(1, -1), lp["ln1_h_b"].reshape(1, -1),
            lp["ffn_h1"]["W"], lp["ffn_h1"]["b"].reshape(1, -1),
            lp["ffn_h2"]["W"], lp["ffn_h2"]["b"].reshape(1, -1),
            lp["ln2_h_g"].reshape(1, -1), lp["ln2_h_b"].reshape(1, -1)]
    if not last:
        args += [lp_next["Q"]["W"], lp_next["Q"]["b"].reshape(1, -1),
                 lp_next["K"]["W"], lp_next["K"]["b"].reshape(1, -1),
                 lp_next["V"]["W"], lp_next["V"]["b"].reshape(1, -1)]
    n_out = 1 if last else 4
    return pl.pallas_call(
        functools.partial(_node_body, last),
        grid=(N_NODES // NBLK,),
        in_specs=[_rows(H, NBLK)] * 5 + [_full(a) for a in args[5:]],
        out_specs=[_rows(H, NBLK)] * n_out,
        out_shape=[jax.ShapeDtypeStruct((N_NODES, H), F32)] * n_out,
    )(*args)


def _sc_gather_body(k_hbm, q_hbm, v_hbm, src2, dst2, kg_hbm, qg_hbm, vg_hbm,
                    idx_s, idx_d,
                    bufk0, bufk1, bufq0, bufq1, bufv0, bufv1,
                    gsk0, gsk1, gsq0, gsq1, gsv0, gsv1,
                    wsk0, wsk1, wsq0, wsq1, wsv0, wsv1):
    cid = lax.axis_index("c")
    sid = lax.axis_index("s")
    wid = cid * 16 + sid
    pltpu.sync_copy(src2.at[wid], idx_s)
    pltpu.sync_copy(dst2.at[wid], idx_d)
    ebase = wid * EPW_G
    bufs = ((bufk0, bufq0, bufv0), (bufk1, bufq1, bufv1))
    gs = ((gsk0, gsq0, gsv0), (gsk1, gsq1, gsv1))
    ws = ((wsk0, wsq0, wsv0), (wsk1, wsq1, wsv1))

    def g_start(cc, b):
        pltpu.async_copy(k_hbm.at[idx_s.at[cc]], bufs[b][0], gs[b][0])
        pltpu.async_copy(q_hbm.at[idx_d.at[cc]], bufs[b][1], gs[b][1])
        pltpu.async_copy(v_hbm.at[idx_s.at[cc]], bufs[b][2], gs[b][2])

    def g_wait(cc, b):
        pltpu.make_async_copy(k_hbm.at[idx_s.at[cc]], bufs[b][0], gs[b][0]).wait()
        pltpu.make_async_copy(q_hbm.at[idx_d.at[cc]], bufs[b][1], gs[b][1]).wait()
        pltpu.make_async_copy(v_hbm.at[idx_s.at[cc]], bufs[b][2], gs[b][2]).wait()

    def w_descr(cc, b):
        row0 = pl.multiple_of(ebase + cc * C_G, 8)
        return ((bufs[b][0], kg_hbm.at[pl.ds(row0, C_G), :], ws[b][0]),
                (bufs[b][1], qg_hbm.at[pl.ds(row0, C_G), :], ws[b][1]),
                (bufs[b][2], vg_hbm.at[pl.ds(row0, C_G), :], ws[b][2]))

    def w_start(cc, b):
        for s, d, sem in w_descr(cc, b):
            pltpu.async_copy(s, d, sem)

    def w_wait(cc, b):
        for s, d, sem in w_descr(cc, b):
            pltpu.make_async_copy(s, d, sem).wait()

    g_start(0, 0)

    def body(g, carry):
        for b in (0, 1):
            cc = 2 * g + b
            nb = (b + 1) % 2

            @pl.when(cc >= 1)
            def _():
                w_wait(cc - 1, nb)

            g_start(cc + 1, nb)
            g_wait(cc, b)
            w_start(cc, b)
        return carry

    lax.fori_loop(0, (NCH_G - 1) // 2, body, 0)
    w_wait(NCH_G - 2, 1)
    g_wait(NCH_G - 1, 0)
    w_start(NCH_G - 1, 0)
    w_wait(NCH_G - 1, 0)


def _sc_scatter_body(sv_hbm, sb_hbm, dst4, zin_hbm, out_hbm,
                     idx_d, bufv0, bufv1, acc_sh, sem0, sem1):
    cid = lax.axis_index("c")
    sid = lax.axis_index("s")
    r0 = pl.multiple_of(sid * STR, 8)
    pltpu.sync_copy(zin_hbm.at[pl.ds(r0, STR), :], acc_sh.at[pl.ds(r0, STR), :])
    rem = N_NODES - 16 * STR

    @pl.when(sid == 0)
    def _():
        pltpu.sync_copy(zin_hbm.at[pl.ds(16 * STR, rem), :],
                        acc_sh.at[pl.ds(16 * STR, rem), :])

    plsc.subcore_barrier()
    ebase = sid * EPW_S
    sems = (sem0, sem1)
    bufs = (bufv0, bufv1)

    def mkpipe(src_hbm):
        def r_start(cc, b):
            row0 = pl.multiple_of(ebase + cc * C_S, 8)
            pltpu.async_copy(src_hbm.at[pl.ds(row0, C_S), :], bufs[b], sems[b])

        def r_wait(cc, b):
            row0 = pl.multiple_of(ebase + cc * C_S, 8)
            pltpu.make_async_copy(src_hbm.at[pl.ds(row0, C_S), :], bufs[b],
                                  sems[b]).wait()

        def consume(cc, b):
            pltpu.sync_copy(bufs[b], acc_sh.at[idx_d.at[lax.rem(cc, IW)]],
                            add=True)

        def refill(cc):
            @pl.when(lax.rem(cc, IW) == 0)
            def _():
                pltpu.sync_copy(dst4.at[sid, lax.div(cc, IW)], idx_d)

        def run():
            r_start(0, 0)

            def body(g, carry):
                for b in (0, 1):
                    cc = 2 * g + b
                    nb = (b + 1) % 2
                    refill(cc)

                    @pl.when(cc + 1 < NCH_S)
                    def _():
                        r_start(cc + 1, nb)

                    r_wait(cc, b)
                    consume(cc, b)
                return carry

            lax.fori_loop(0, (NCH_S - 1) // 2, body, 0)
            cc = NCH_S - 1
            refill(cc)
            r_wait(cc, cc % 2)
            consume(cc, cc % 2)
        return run

    @pl.when(cid == 0)
    def _():
        mkpipe(sv_hbm)()

    @pl.when(cid == 1)
    def _():
        mkpipe(sb_hbm)()

    plsc.subcore_barrier()
    pltpu.sync_copy(acc_sh.at[pl.ds(r0, STR), :],
                    out_hbm.at[cid, pl.ds(r0, STR), :])

    @pl.when(sid == 0)
    def _():
        pltpu.sync_copy(acc_sh.at[pl.ds(16 * STR, rem), :],
                        out_hbm.at[cid, pl.ds(16 * STR, rem), :])


@functools.lru_cache(maxsize=None)
def _sc_kernels():
    mesh = plsc.VectorSubcoreMesh(core_axis_name="c", subcore_axis_name="s")
    gather = pl.kernel(
        _sc_gather_body, mesh=mesh,
        out_type=[jax.ShapeDtypeStruct((HE, H), F32)] * 3,
        scratch_types=[pltpu.VMEM((NCH_G, C_G), jnp.int32),
                       pltpu.VMEM((NCH_G, C_G), jnp.int32)]
                      + [pltpu.VMEM((C_G, H), F32) for _ in range(6)]
                      + [pltpu.SemaphoreType.DMA for _ in range(12)],
    )
    scatter = pl.kernel(
        _sc_scatter_body, mesh=mesh,
        out_type=[jax.ShapeDtypeStruct((2, N_NODES, H), F32)],
        scratch_types=[pltpu.VMEM((IW, C_S), jnp.int32),
                       pltpu.VMEM((C_S, H), F32),
                       pltpu.VMEM((C_S, H), F32),
                       pltpu.VMEM_SHARED((N_NODES, H), F32),
                       pltpu.SemaphoreType.DMA,
                       pltpu.SemaphoreType.DMA],
    )
    return gather, scatter


def _gather_kqv(k, q, v, src2, dst2):
    return _sc_kernels()[0](k, q, v, src2, dst2)


def _scatter_segments(sv, sb, dst4):
    zin = jnp.zeros((N_NODES, H), F32)
    return _sc_kernels()[1](sv, sb, dst4, zin)[0]


def kernel(x, lap_pos_enc, edge_attr, params, edge_index):
    src_h = [edge_index[0, i * HE:(i + 1) * HE].reshape(NW, NCH_G, C_G)
             for i in range(HALVES)]
    dst_h = [edge_index[1, i * HE:(i + 1) * HE].reshape(NW, NCH_G, C_G)
             for i in range(HALVES)]
    dst4_h = [edge_index[1, i * HE:(i + 1) * HE].reshape(16, NCH_S // IW,
                                                         IW, C_S)
              for i in range(HALVES)]
    h, q, k, v = _node_init(x, lap_pos_enc, params)
    e = [edge_attr] * HALVES
    for li in range(4):
        lp = params["layers"][li]
        lp_next = params["layers"][li + 1] if li < 3 else None
        g = [_gather_kqv(k, q, v, src_h[i], dst_h[i]) for i in range(HALVES)]
        wvz = []
        e_next = [None] * HALVES
        for i in range(HALVES):
            outs = _edge_pass(li, i, lp, e[i], *g[i], params)
            if li < 3:
                sv, sb, e_next[i] = outs
            else:
                sv, sb = outs
            wvz.append(_scatter_segments(sv, sb, dst4_h[i]))
        e = e_next
        nouts = _node_pass(li, lp, lp_next, h, wvz[0], wvz[1])
        if li < 3:
            h, q, k, v = nouts
        else:
            h = nouts[0]
    return h
